# Initial kernel scaffold; baseline (speedup 1.0000x reference)
import jax
import jax.numpy as jnp
from jax.experimental import pallas as pl


def kernel(x, index):
    raise NotImplementedError("write your pallas kernel here")


# speedup vs baseline: 135.5224x; 135.5224x over previous
import functools

import jax
import jax.numpy as jnp
from jax import lax
from jax.experimental import pallas as pl
from jax.experimental.pallas import tpu as pltpu
from jax.experimental.pallas import tpu_sc as plsc

_ROWS = 16384
_COLS = 200
_B = _ROWS * _COLS
_NC = 2
_NS = 16
_NW = _NC * _NS
_PW = _B // _NW
_NCHUNK = 8
_C = _PW // _NCHUNK


def _make_gather():
    mesh = plsc.VectorSubcoreMesh(core_axis_name="c", subcore_axis_name="s")

    @functools.partial(
        pl.kernel,
        out_type=jax.ShapeDtypeStruct((_B,), jnp.float32),
        mesh=mesh,
        scratch_types=[
            pltpu.VMEM((_C,), jnp.int32),
            pltpu.VMEM((_C,), jnp.float32),
            pltpu.SemaphoreType.DMA,
        ],
    )
    def gather_kernel(x_hbm, idx_hbm, out_hbm, idx_v, rows_v, sem):
        wid = lax.axis_index("s") * _NC + lax.axis_index("c")
        base = wid * _PW
        for c in range(_NCHUNK):
            off = base + c * _C
            pltpu.sync_copy(idx_hbm.at[pl.ds(off, _C)], idx_v)
            pltpu.async_copy(x_hbm.at[idx_v], rows_v, sem).wait()
            pltpu.sync_copy(rows_v, out_hbm.at[pl.ds(off, _C)])

    return gather_kernel


_gather = _make_gather()


@jax.jit
def kernel(x, index):
    idx_flat = index.reshape(-1).astype(jnp.int32)
    out = _gather(x, idx_flat)
    return out.reshape(index.shape)


# speedup vs baseline: 139.5448x; 1.0297x over previous
import functools

import jax
import jax.numpy as jnp
from jax import lax
from jax.experimental import pallas as pl
from jax.experimental.pallas import tpu as pltpu
from jax.experimental.pallas import tpu_sc as plsc

_ROWS = 16384
_COLS = 200
_B = _ROWS * _COLS
_NC = 2
_NS = 16
_NW = _NC * _NS
_PW = _B // _NW
_NCHUNK = 8
_C = _PW // _NCHUNK


def _make_gather():
    mesh = plsc.VectorSubcoreMesh(core_axis_name="c", subcore_axis_name="s")

    @functools.partial(
        pl.kernel,
        out_type=jax.ShapeDtypeStruct((_B,), jnp.float32),
        mesh=mesh,
        scratch_types=[
            pltpu.VMEM((_C,), jnp.int32),
            pltpu.VMEM((_C,), jnp.int32),
            pltpu.VMEM((_C,), jnp.float32),
            pltpu.VMEM((_C,), jnp.float32),
            pltpu.SemaphoreType.DMA,
            pltpu.SemaphoreType.DMA,
            pltpu.SemaphoreType.DMA,
            pltpu.SemaphoreType.DMA,
            pltpu.SemaphoreType.DMA,
            pltpu.SemaphoreType.DMA,
        ],
    )
    def gather_kernel(x_hbm, idx_hbm, out_hbm,
                      i0, i1, r0, r1, si0, si1, sg0, sg1, so0, so1):
        idx_bufs, row_bufs = (i0, i1), (r0, r1)
        si, sg, so = (si0, si1), (sg0, sg1), (so0, so1)
        wid = lax.axis_index("s") * _NC + lax.axis_index("c")
        base = wid * _PW

        def idx_cp(c):
            b = c % 2
            return pltpu.make_async_copy(
                idx_hbm.at[pl.ds(base + c * _C, _C)], idx_bufs[b], si[b])

        def gather_cp(c):
            b = c % 2
            return pltpu.make_async_copy(
                x_hbm.at[idx_bufs[b]], row_bufs[b], sg[b])

        def out_cp(c):
            b = c % 2
            return pltpu.make_async_copy(
                row_bufs[b], out_hbm.at[pl.ds(base + c * _C, _C)], so[b])

        idx_cp(0).start()
        idx_cp(1).start()
        for c in range(_NCHUNK):
            idx_cp(c).wait()
            if c >= 2:
                out_cp(c - 2).wait()
            gather_cp(c).start()
            if c >= 1:
                gather_cp(c - 1).wait()
                out_cp(c - 1).start()
                if c + 1 < _NCHUNK:
                    idx_cp(c + 1).start()
        gather_cp(_NCHUNK - 1).wait()
        out_cp(_NCHUNK - 1).start()
        out_cp(_NCHUNK - 2).wait()
        out_cp(_NCHUNK - 1).wait()

    return gather_kernel


_gather = _make_gather()


@jax.jit
def kernel(x, index):
    idx_flat = index.reshape(-1).astype(jnp.int32)
    out = _gather(x, idx_flat)
    return out.reshape(index.shape)


# speedup vs baseline: 225.8656x; 1.6186x over previous
import functools

import jax
import jax.numpy as jnp
from jax import lax
from jax.experimental import pallas as pl
from jax.experimental.pallas import tpu as pltpu
from jax.experimental.pallas import tpu_sc as plsc

_ROWS = 16384
_COLS = 200
_B = _ROWS * _COLS
_NC = 2
_NS = 16
_NW = _NC * _NS
_PW = _B // _NW
_NCHUNK = 8
_C = _PW // _NCHUNK
_V = 1000000
_STAGE = 62504


def _make_gather():
    mesh = plsc.VectorSubcoreMesh(core_axis_name="c", subcore_axis_name="s")

    @functools.partial(
        pl.kernel,
        out_type=jax.ShapeDtypeStruct((_B,), jnp.float32),
        mesh=mesh,
        scratch_types=[
            pltpu.VMEM((_C,), jnp.int32),
            pltpu.VMEM((_C,), jnp.int32),
            pltpu.VMEM((_C,), jnp.float32),
            pltpu.VMEM((_C,), jnp.float32),
            pltpu.SemaphoreType.DMA,
            pltpu.SemaphoreType.DMA,
            pltpu.SemaphoreType.DMA,
            pltpu.SemaphoreType.DMA,
            pltpu.SemaphoreType.DMA,
            pltpu.SemaphoreType.DMA,
            pltpu.VMEM_SHARED((_V,), jnp.float32),
        ],
    )
    def gather_kernel(x_hbm, idx_hbm, out_hbm,
                      i0, i1, r0, r1, si0, si1, sg0, sg1, so0, so1,
                      table_sh):
        idx_bufs, row_bufs = (i0, i1), (r0, r1)
        si, sg, so = (si0, si1), (sg0, sg1), (so0, so1)
        sid = lax.axis_index("s")
        wid = sid * _NC + lax.axis_index("c")
        base = wid * _PW

        stage_off = sid * _STAGE

        def stage(off, size):
            pltpu.sync_copy(x_hbm.at[pl.ds(off, size)], r0.at[pl.ds(0, size)])
            pltpu.sync_copy(r0.at[pl.ds(0, size)], table_sh.at[pl.ds(off, size)])

        _SCH = 12504
        for k in range(4):
            stage(stage_off + k * _SCH, _SCH)

        @pl.when(sid < _NS - 1)
        def _():
            stage(stage_off + 4 * _SCH, _STAGE - 4 * _SCH)

        @pl.when(sid == _NS - 1)
        def _():
            tail = _V - (_NS - 1) * _STAGE
            stage(stage_off + 4 * _SCH, tail - 4 * _SCH)

        plsc.subcore_barrier()

        def idx_cp(c):
            b = c % 2
            return pltpu.make_async_copy(
                idx_hbm.at[pl.ds(base + c * _C, _C)], idx_bufs[b], si[b])

        def gather_cp(c):
            b = c % 2
            return pltpu.make_async_copy(
                table_sh.at[idx_bufs[b]], row_bufs[b], sg[b])

        def out_cp(c):
            b = c % 2
            return pltpu.make_async_copy(
                row_bufs[b], out_hbm.at[pl.ds(base + c * _C, _C)], so[b])

        idx_cp(0).start()
        idx_cp(1).start()
        for c in range(_NCHUNK):
            idx_cp(c).wait()
            if c >= 2:
                out_cp(c - 2).wait()
            gather_cp(c).start()
            if c >= 1:
                gather_cp(c - 1).wait()
                out_cp(c - 1).start()
                if c + 1 < _NCHUNK:
                    idx_cp(c + 1).start()
        gather_cp(_NCHUNK - 1).wait()
        out_cp(_NCHUNK - 1).start()
        out_cp(_NCHUNK - 2).wait()
        out_cp(_NCHUNK - 1).wait()

    return gather_kernel


_gather = _make_gather()


@jax.jit
def kernel(x, index):
    idx_flat = index.reshape(-1).astype(jnp.int32)
    out = _gather(x, idx_flat)
    return out.reshape(index.shape)
